# baseline (speedup 1.0000x reference)
import functools

import jax
import jax.numpy as jnp
from jax import lax
from jax.experimental import pallas as pl
from jax.experimental.pallas import tpu as pltpu
from jax.experimental.pallas import tpu_sc as plsc


def _proj_body(x_ref, w_ref, b_ref, o_ref):
    y = (
        jnp.dot(x_ref[...], w_ref[...], preferred_element_type=jnp.float32)
        + b_ref[0:1, :]
    )
    o_ref[...] = y.reshape(o_ref.shape)


def _project_table_dup(word_table, W, b, pack=4, blk=2000):
    V, D = word_table.shape
    wd = jnp.concatenate([W.T, W.T], axis=1)
    w_bd = jnp.kron(jnp.eye(pack, dtype=W.dtype), wd)
    b_rep = jnp.broadcast_to(
        jnp.tile(jnp.concatenate([b, b]), pack)[None, :], (8, pack * 2 * D)
    )
    x = word_table.reshape(V // pack, pack * D)
    n_rows = V // pack
    proj = pl.pallas_call(
        _proj_body,
        grid=(n_rows // blk,),
        in_specs=[
            pl.BlockSpec((blk, pack * D), lambda i: (i, 0)),
            pl.BlockSpec((pack * D, pack * 2 * D), lambda i: (0, 0)),
            pl.BlockSpec((8, pack * 2 * D), lambda i: (0, 0)),
        ],
        out_specs=pl.BlockSpec((pack * blk, 2 * D), lambda i: (i, 0)),
        out_shape=jax.ShapeDtypeStruct((V, 2 * D), jnp.float32),
    )(x, w_bd, b_rep)
    return proj


def _make_sc_gather_t(V, L, B, D):
    NW = 32
    SUB = B // NW
    assert SUB == 128 and D == 64

    mesh = plsc.VectorSubcoreMesh(core_axis_name="c", subcore_axis_name="s")

    @functools.partial(
        pl.kernel,
        out_type=jax.ShapeDtypeStruct((L, D, B), jnp.float32),
        mesh=mesh,
        scratch_types=[
            pltpu.VMEM((L, SUB), jnp.int32),
            pltpu.VMEM((2, SUB, 2 * D), jnp.float32),
            pltpu.VMEM((2, D, SUB), jnp.float32),
            pltpu.SemaphoreType.DMA,
            pltpu.SemaphoreType.DMA,
            pltpu.SemaphoreType.DMA,
            pltpu.SemaphoreType.DMA,
        ],
        compiler_params=pltpu.CompilerParams(
            use_tc_tiling_on_sc=True, needs_layout_passes=False
        ),
    )
    def sc_gather(ptab_hbm, idxt_hbm, out_hbm, idx_v, gbuf, tbuf,
                  g0, g1, w0, w1):
        wid = lax.axis_index("s") * 2 + lax.axis_index("c")
        bb = wid * SUB
        pltpu.sync_copy(idxt_hbm.at[:, pl.ds(bb, SUB)], idx_v)
        gsems = (g0, g1)
        wsems = (w0, w1)
        lane = lax.iota(jnp.int32, 16)

        def fire(l, s):
            return pltpu.async_copy(
                ptab_hbm.at[idx_v.at[l]], gbuf.at[s], gsems[s])

        def transpose_into(s):
            @pl.loop(0, D)
            def _d(d):
                dcol = jnp.full((16,), 0, jnp.int32) + d
                for j in range(SUB // 16):
                    rows = lane + (j * 16)
                    vals = plsc.load_gather(gbuf.at[s], [rows, dcol])
                    tbuf[s, d, pl.ds(j * 16, 16)] = vals

        h0 = fire(0, 0)
        h1 = fire(1, 1)
        handles = [h0, h1]

        @pl.loop(0, L // 2)
        def _l(i):
            for s in range(2):
                l = i * 2 + s
                handles[s].wait()

                @pl.when(i > 0)
                def _drain_prev_write():
                    pltpu.make_async_copy(
                        tbuf.at[s], out_hbm.at[l - 2, :, pl.ds(bb, SUB)],
                        wsems[s]).wait()

                transpose_into(s)

                @pl.when(i < L // 2 - 1)
                def _refire():
                    fire(l + 2, s)

                pltpu.async_copy(
                    tbuf.at[s], out_hbm.at[l, :, pl.ds(bb, SUB)], wsems[s])

        pltpu.make_async_copy(
            tbuf.at[0], out_hbm.at[L - 2, :, pl.ds(bb, SUB)], wsems[0]).wait()
        pltpu.make_async_copy(
            tbuf.at[1], out_hbm.at[L - 1, :, pl.ds(bb, SUB)], wsems[1]).wait()

    return sc_gather


def kernel(inputs, word_table, W, b):
    V, D = word_table.shape
    B, L = inputs.shape
    ptab = _project_table_dup(word_table, W, b)
    idx_t = inputs.T
    out_t = _make_sc_gather_t(V, L, B, D)(ptab, idx_t)
    return out_t.transpose(2, 0, 1)


# speedup vs baseline: 1.2814x; 1.2814x over previous
import functools

import jax
import jax.numpy as jnp
from jax import lax
from jax.experimental import pallas as pl
from jax.experimental.pallas import tpu as pltpu
from jax.experimental.pallas import tpu_sc as plsc


def _proj_body(x_ref, w_ref, b_ref, o_ref):
    o_ref[...] = (
        jnp.dot(x_ref[...], w_ref[...], preferred_element_type=jnp.float32)
        + b_ref[0:1, :]
    )


def _project_table(word_table, W, b, pack=4, blk=2000):
    V, D = word_table.shape
    w_bd = jnp.kron(jnp.eye(pack, dtype=W.dtype), W.T)
    b_rep = jnp.broadcast_to(jnp.tile(b, pack)[None, :], (8, pack * D))
    x = word_table.reshape(V // pack, pack * D)
    n_rows = V // pack
    proj = pl.pallas_call(
        _proj_body,
        grid=(n_rows // blk,),
        in_specs=[
            pl.BlockSpec((blk, pack * D), lambda i: (i, 0)),
            pl.BlockSpec((pack * D, pack * D), lambda i: (0, 0)),
            pl.BlockSpec((8, pack * D), lambda i: (0, 0)),
        ],
        out_specs=pl.BlockSpec((blk, pack * D), lambda i: (i, 0)),
        out_shape=jax.ShapeDtypeStruct((n_rows, pack * D), jnp.float32),
    )(x, w_bd, b_rep)
    return proj.reshape(V, D)


_GL = 128


def _make_sc_gather(V, D, NW, per_w, ch):
    n_grp = per_w // _GL
    g_per_ch = ch // _GL
    n_pairs = per_w // (2 * ch)
    assert n_pairs * 2 * ch == per_w

    mesh = plsc.VectorSubcoreMesh(core_axis_name="c", subcore_axis_name="s")

    @functools.partial(
        pl.kernel,
        out_type=jax.ShapeDtypeStruct((NW * per_w, D), jnp.float32),
        mesh=mesh,
        scratch_types=[
            pltpu.VMEM((n_grp, _GL), jnp.int32),
            pltpu.VMEM((2, ch, D), jnp.float32),
            pltpu.SemaphoreType.DMA,
            pltpu.SemaphoreType.DMA,
        ],
        compiler_params=pltpu.CompilerParams(use_tc_tiling_on_sc=False),
    )
    def sc_gather(ptab_hbm, idx_hbm, out_hbm, idx_v, rows_v, sem0, sem1):
        wid = lax.axis_index("s") * 2 + lax.axis_index("c")
        base = wid * per_w
        pltpu.sync_copy(idx_hbm.at[wid], idx_v)
        sems = (sem0, sem1)

        @pl.loop(0, n_pairs)
        def _pair(i):
            handles = ([], [])
            for bb in range(2):
                cidx = i * 2 + bb
                for g in range(g_per_ch):
                    row = cidx * g_per_ch + g
                    handles[bb].append(
                        pltpu.async_copy(
                            ptab_hbm.at[idx_v.at[row]],
                            rows_v.at[bb, pl.ds(g * _GL, _GL)],
                            sems[bb],
                        )
                    )
            for bb in range(2):
                for h in handles[bb]:
                    h.wait()
                cidx = i * 2 + bb
                pltpu.sync_copy(
                    rows_v.at[bb], out_hbm.at[pl.ds(base + cidx * ch, ch)]
                )

    return sc_gather


def kernel(inputs, word_table, W, b):
    V, D = word_table.shape
    B, L = inputs.shape
    ptab = _project_table(word_table, W, b)

    NW = 32
    total = B * L
    per_w = total // NW
    idx3 = inputs.reshape(NW, per_w // _GL, _GL)
    out2 = _make_sc_gather(V, D, NW, per_w, ch=512)(ptab, idx3)
    return out2.reshape(B, L, D)
